# baseline (speedup 1.0000x reference)
import functools

import jax
import jax.numpy as jnp
from jax import lax
from jax.experimental import pallas as pl
from jax.experimental.pallas import tpu as pltpu
from jax.experimental.pallas import tpu_sc as plsc

B, N, K = 64, 32768, 512
NC, NS, L = 2, 16, 16
NW = NC * NS
RW = B // NW

_mesh = plsc.VectorSubcoreMesh(core_axis_name="c", subcore_axis_name="s")


@functools.partial(
    pl.kernel,
    mesh=_mesh,
    out_type=jax.ShapeDtypeStruct((B, K), jnp.float32),
    scratch_types=[
        pltpu.VMEM((K,), jnp.float32),
        pltpu.SemaphoreType.DMA,
    ],
    compiler_params=pltpu.CompilerParams(needs_layout_passes=False),
)
def _probe(x_hbm, mv_hbm, out_hbm, buf, sem):
    wid = lax.axis_index("s") * NC + lax.axis_index("c")
    row0 = wid * RW
    pltpu.async_copy(x_hbm.at[row0, pl.ds(0, K)], buf, sem).wait()
    pltpu.async_copy(buf, out_hbm.at[row0], sem).wait()


def kernel(x, possible_moves):
    return _probe(x, possible_moves.astype(jnp.int32))
